# baseline (speedup 1.0000x reference)
import functools

import jax
import jax.numpy as jnp
from jax import lax
from jax.experimental import pallas as pl
from jax.experimental.pallas import tpu as pltpu
from jax.experimental.pallas import tpu_sc as plsc

H = 4
K = 8
DK = 64
NKEYS = 128
D = 1024
T = 2048
TB = 256

NC = 2
NS = 16
NW = NC * NS
TPW = T // NW
HK = H * K

_NEG = float("-inf")


def _topk8(s, payload=None):
    n = s.shape[1]
    iota = lax.broadcasted_iota(jnp.int32, s.shape, 1)
    vals, idxs = [], []
    cur = s
    for _ in range(K):
        m = jnp.max(cur, axis=1, keepdims=True)
        arg = jnp.min(jnp.where(cur == m, iota, n), axis=1, keepdims=True)
        vals.append(m)
        if payload is None:
            idxs.append(arg)
        else:
            idxs.append(jnp.sum(jnp.where(iota == arg, payload, 0),
                                axis=1, keepdims=True))
        cur = jnp.where(iota == arg, _NEG, cur)
    return jnp.concatenate(vals, axis=1), jnp.concatenate(idxs, axis=1)


def _routing_body(x_ref, wq_ref, k1_ref, k2_ref, idx_ref, score_ref):
    x = x_ref[...]
    q = lax.dot_general(x, wq_ref[...], (((1,), (1,)), ((), ())),
                        preferred_element_type=jnp.float32)
    for h in range(H):
        q1 = q[:, h * 2 * DK: h * 2 * DK + DK]
        q2 = q[:, h * 2 * DK + DK: (h + 1) * 2 * DK]
        sim1 = lax.dot_general(q1, k1_ref[h], (((1,), (1,)), ((), ())),
                               preferred_element_type=jnp.float32)
        sim2 = lax.dot_general(q2, k2_ref[h], (((1,), (1,)), ((), ())),
                               preferred_element_type=jnp.float32)
        s1, i1 = _topk8(sim1)
        s2, i2 = _topk8(sim2)
        all_s = jnp.concatenate([s1[:, i:i + 1] + s2 for i in range(K)], axis=1)
        all_i = jnp.concatenate(
            [i1[:, i:i + 1] * NKEYS + i2 for i in range(K)], axis=1)
        fs, fi = _topk8(all_s, payload=all_i)
        idx_ref[:, h * K:(h + 1) * K] = fi
        score_ref[:, h * K:(h + 1) * K] = jnp.maximum(fs, 0.0)


def _routing(xs, Wq, keys1, keys2):
    grid = (T // TB,)
    return pl.pallas_call(
        _routing_body,
        grid=grid,
        in_specs=[
            pl.BlockSpec((TB, D), lambda i: (i, 0)),
            pl.BlockSpec((2 * H * DK, D), lambda i: (0, 0)),
            pl.BlockSpec((H, NKEYS, DK), lambda i: (0, 0, 0)),
            pl.BlockSpec((H, NKEYS, DK), lambda i: (0, 0, 0)),
        ],
        out_specs=[
            pl.BlockSpec((TB, HK), lambda i: (i, 0)),
            pl.BlockSpec((TB, HK), lambda i: (i, 0)),
        ],
        out_shape=[
            jax.ShapeDtypeStruct((T, HK), jnp.int32),
            jax.ShapeDtypeStruct((T, HK), jnp.float32),
        ],
    )(xs, Wq, keys1, keys2)


def _expert_body(x_hbm, idx_hbm, score_hbm, down_hbm, up_hbm, out_hbm,
                 idx_v, score_v, xrow, d0, d1, u0, u1, obuf, red_v,
                 sem_d0, sem_d1, sem_u0, sem_u1, sem_x):
    wid = lax.axis_index("s") * NC + lax.axis_index("c")
    base = wid * TPW
    pltpu.sync_copy(idx_hbm.at[pl.ds(base, TPW)], idx_v)
    pltpu.sync_copy(score_hbm.at[pl.ds(base, TPW)], score_v)

    zeros16 = jnp.zeros((16,), jnp.float32)
    bufs = (d0, d1, u0, u1)
    sems = (sem_d0, sem_d1, sem_u0, sem_u1)
    tabs = (down_hbm, down_hbm, up_hbm, up_hbm)

    def issue(i, p):
        half = (p % 2) * 16
        pltpu.async_copy(
            tabs[p].at[idx_v.at[i, pl.ds(half, 16)]], bufs[p], sems[p])

    def wait(p):
        pltpu.make_async_copy(
            tabs[p].at[idx_v.at[0, pl.ds((p % 2) * 16, 16)]],
            bufs[p], sems[p]).wait()

    def dots32(par, i):
        def dot_step(c, accs):
            xv = xrow[par, pl.ds(c, 16)]
            lo = [accs[r] + xv * d0[r, pl.ds(c, 16)] for r in range(16)]
            hi = [accs[16 + r] + xv * d1[r, pl.ds(c, 16)] for r in range(16)]
            return tuple(lo + hi)

        accs = plsc.parallel_loop(
            0, D, step=16, unroll=4, carry=(zeros16,) * HK)(dot_step)
        for r in range(HK):
            red_v[r] = accs[r]
        iota16 = lax.iota(jnp.int32, 16)
        hvs = []
        for half in range(2):
            hv = zeros16
            for c in range(16):
                hv = hv + plsc.load_gather(
                    red_v, [iota16 + half * 16, jnp.full((16,), c, jnp.int32)])
            hv = hv / (1.0 + jnp.exp(-hv))
            hvs.append(hv * score_v[i, pl.ds(half * 16, 16)])
        return hvs

    def accum32(hv0, hv1):
        hs0 = [hv0[r] for r in range(16)]
        hs1 = [hv1[r] for r in range(16)]

        def out_step(c, _):
            acc = zeros16
            for r in range(16):
                acc = acc + hs0[r] * u0[r, pl.ds(c, 16)]
            for r in range(16):
                acc = acc + hs1[r] * u1[r, pl.ds(c, 16)]
            obuf[pl.ds(c, 16)] = acc
            return _

        plsc.parallel_loop(0, D, step=16, unroll=4, carry=jnp.int32(0))(
            out_step)

    pltpu.async_copy(x_hbm.at[base], xrow.at[0], sem_x)
    for p in range(4):
        issue(jnp.int32(0), p)

    def pair_step(ii, carry):
        for par in range(2):
            i = ii * 2 + par
            t = base + i
            inext = jnp.minimum(i + 1, TPW - 1)
            pltpu.make_async_copy(x_hbm.at[t], xrow.at[par], sem_x).wait()
            wait(0)
            wait(1)
            hv0, hv1 = dots32(par, i)
            wait(2)
            wait(3)
            accum32(hv0, hv1)
            pltpu.async_copy(x_hbm.at[base + inext], xrow.at[1 - par], sem_x)
            for p in range(4):
                issue(inext, p)
            pltpu.sync_copy(obuf, out_hbm.at[t])
        return carry

    lax.fori_loop(0, TPW // 2, pair_step, 0)
    pltpu.make_async_copy(x_hbm.at[base], xrow.at[0], sem_x).wait()
    for p in range(4):
        wait(p)


def _expert(xs, idx, score, expert_down, expert_up):
    mesh = plsc.VectorSubcoreMesh(core_axis_name="c", subcore_axis_name="s")
    run = pl.kernel(
        _expert_body, mesh=mesh,
        compiler_params=pltpu.CompilerParams(needs_layout_passes=False),
        out_type=jax.ShapeDtypeStruct((T, D), jnp.float32),
        scratch_types=[
            pltpu.VMEM((TPW, HK), jnp.int32),
            pltpu.VMEM((TPW, HK), jnp.float32),
            pltpu.VMEM((2, D), jnp.float32),
            pltpu.VMEM((16, D), jnp.float32),
            pltpu.VMEM((16, D), jnp.float32),
            pltpu.VMEM((16, D), jnp.float32),
            pltpu.VMEM((16, D), jnp.float32),
            pltpu.VMEM((D,), jnp.float32),
            pltpu.VMEM((16, 16), jnp.float32),
            pltpu.SemaphoreType.DMA,
            pltpu.SemaphoreType.DMA,
            pltpu.SemaphoreType.DMA,
            pltpu.SemaphoreType.DMA,
            pltpu.SemaphoreType.DMA,
        ],
    )
    return run(xs, idx, score, expert_down, expert_up)


def kernel(x, Wq, keys, expert_down, expert_up):
    xs = x[0]
    keys1 = keys[:, :, 0, :]
    keys2 = keys[:, :, 1, :]
    idx, score = _routing(xs, Wq, keys1, keys2)
    out = _expert(xs, idx, score, expert_down, expert_up)
    return out[None]


# speedup vs baseline: 1.1133x; 1.1133x over previous
import functools

import jax
import jax.numpy as jnp
from jax import lax
from jax.experimental import pallas as pl
from jax.experimental.pallas import tpu as pltpu
from jax.experimental.pallas import tpu_sc as plsc

H = 4
K = 8
DK = 64
NKEYS = 128
D = 1024
T = 2048
TB = 256

NC = 2
NS = 16
NW = NC * NS
TPW = T // NW
HK = H * K

_NEG = float("-inf")


def _topk8_packed(s, nbits):
    iota = lax.broadcasted_iota(jnp.int32, s.shape, 1)
    mask = jnp.int32(~((1 << nbits) - 1))
    bits = lax.bitcast_convert_type(s, jnp.int32)
    cur = lax.bitcast_convert_type((bits & mask) | iota, jnp.float32)
    vals = []
    for _ in range(K):
        m = jnp.max(cur, axis=1, keepdims=True)
        vals.append(m)
        cur = jnp.where(cur == m, _NEG, cur)
    return jnp.concatenate(vals, axis=1)


def _routing_body(x_ref, wq_ref, k1_ref, k2_ref, idx_ref, score_ref):
    x = x_ref[...]
    q = lax.dot_general(x, wq_ref[...], (((1,), (1,)), ((), ())),
                        preferred_element_type=jnp.float32)
    for h in range(H):
        q1 = q[:, h * 2 * DK: h * 2 * DK + DK]
        q2 = q[:, h * 2 * DK + DK: (h + 1) * 2 * DK]
        sim1 = lax.dot_general(q1, k1_ref[h], (((1,), (1,)), ((), ())),
                               preferred_element_type=jnp.float32)
        sim2 = lax.dot_general(q2, k2_ref[h], (((1,), (1,)), ((), ())),
                               preferred_element_type=jnp.float32)
        p1 = _topk8_packed(sim1, 7)
        p2 = _topk8_packed(sim2, 7)
        i1 = lax.bitcast_convert_type(p1, jnp.int32) & (NKEYS - 1)
        i2 = lax.bitcast_convert_type(p2, jnp.int32) & (NKEYS - 1)
        all_s = jnp.concatenate([p1[:, i:i + 1] + p2 for i in range(K)], axis=1)
        pf = _topk8_packed(all_s, 6)
        pos = lax.bitcast_convert_type(pf, jnp.int32) & 63
        ipos = pos >> 3
        jpos = pos & 7
        iota8 = lax.broadcasted_iota(jnp.int32, i1.shape, 1)
        fi = []
        for s_ in range(K):
            sel1 = jnp.sum(jnp.where(iota8 == ipos[:, s_:s_ + 1], i1, 0),
                           axis=1, keepdims=True)
            sel2 = jnp.sum(jnp.where(iota8 == jpos[:, s_:s_ + 1], i2, 0),
                           axis=1, keepdims=True)
            fi.append(sel1 * NKEYS + sel2)
        idx_ref[:, h * K:(h + 1) * K] = jnp.concatenate(fi, axis=1)
        score_ref[:, h * K:(h + 1) * K] = jnp.maximum(pf, 0.0)


def _routing(xs, Wq, keys1, keys2):
    grid = (T // TB,)
    return pl.pallas_call(
        _routing_body,
        grid=grid,
        in_specs=[
            pl.BlockSpec((TB, D), lambda i: (i, 0)),
            pl.BlockSpec((2 * H * DK, D), lambda i: (0, 0)),
            pl.BlockSpec((H, NKEYS, DK), lambda i: (0, 0, 0)),
            pl.BlockSpec((H, NKEYS, DK), lambda i: (0, 0, 0)),
        ],
        out_specs=[
            pl.BlockSpec((TB, HK), lambda i: (i, 0)),
            pl.BlockSpec((TB, HK), lambda i: (i, 0)),
        ],
        out_shape=[
            jax.ShapeDtypeStruct((T, HK), jnp.int32),
            jax.ShapeDtypeStruct((T, HK), jnp.float32),
        ],
    )(xs, Wq, keys1, keys2)


def _expert_body(x_hbm, idx_hbm, score_hbm, down_hbm, up_hbm, out_hbm,
                 idx_v, score_v, xrow, d0, d1, u0, u1, obuf, red_v,
                 sem_d0, sem_d1, sem_u0, sem_u1, sem_x):
    wid = lax.axis_index("s") * NC + lax.axis_index("c")
    base = wid * TPW
    pltpu.sync_copy(idx_hbm.at[pl.ds(base, TPW)], idx_v)
    pltpu.sync_copy(score_hbm.at[pl.ds(base, TPW)], score_v)

    zeros16 = jnp.zeros((16,), jnp.float32)
    bufs = (d0, d1, u0, u1)
    sems = (sem_d0, sem_d1, sem_u0, sem_u1)
    tabs = (down_hbm, down_hbm, up_hbm, up_hbm)

    def issue(i, p):
        half = (p % 2) * 16
        pltpu.async_copy(
            tabs[p].at[idx_v.at[i, pl.ds(half, 16)]], bufs[p], sems[p])

    def wait(p):
        pltpu.make_async_copy(
            tabs[p].at[idx_v.at[0, pl.ds((p % 2) * 16, 16)]],
            bufs[p], sems[p]).wait()

    def dots32(par, i):
        def dot_step(c, accs):
            xv = xrow[par, pl.ds(c, 16)]
            lo = [accs[r] + xv * d0[r, pl.ds(c, 16)] for r in range(16)]
            hi = [accs[16 + r] + xv * d1[r, pl.ds(c, 16)] for r in range(16)]
            return tuple(lo + hi)

        accs = plsc.parallel_loop(
            0, D, step=16, unroll=4, carry=(zeros16,) * HK)(dot_step)
        for r in range(HK):
            red_v[r] = accs[r]
        iota16 = lax.iota(jnp.int32, 16)
        hvs = []
        for half in range(2):
            hv = zeros16
            for c in range(16):
                hv = hv + plsc.load_gather(
                    red_v, [iota16 + half * 16, jnp.full((16,), c, jnp.int32)])
            hv = hv / (1.0 + jnp.exp(-hv))
            hvs.append(hv * score_v[i, pl.ds(half * 16, 16)])
        return hvs

    def accum32(hv0, hv1):
        hs0 = [hv0[r] for r in range(16)]
        hs1 = [hv1[r] for r in range(16)]

        def out_step(c, _):
            acc = zeros16
            for r in range(16):
                acc = acc + hs0[r] * u0[r, pl.ds(c, 16)]
            for r in range(16):
                acc = acc + hs1[r] * u1[r, pl.ds(c, 16)]
            obuf[pl.ds(c, 16)] = acc
            return _

        plsc.parallel_loop(0, D, step=16, unroll=4, carry=jnp.int32(0))(
            out_step)

    pltpu.async_copy(x_hbm.at[base], xrow.at[0], sem_x)
    for p in range(4):
        issue(jnp.int32(0), p)

    def pair_step(ii, carry):
        for par in range(2):
            i = ii * 2 + par
            t = base + i
            inext = jnp.minimum(i + 1, TPW - 1)
            pltpu.make_async_copy(x_hbm.at[t], xrow.at[par], sem_x).wait()
            wait(0)
            wait(1)
            hv0, hv1 = dots32(par, i)
            wait(2)
            wait(3)
            accum32(hv0, hv1)
            pltpu.async_copy(x_hbm.at[base + inext], xrow.at[1 - par], sem_x)
            for p in range(4):
                issue(inext, p)
            pltpu.sync_copy(obuf, out_hbm.at[t])
        return carry

    lax.fori_loop(0, TPW // 2, pair_step, 0)
    pltpu.make_async_copy(x_hbm.at[base], xrow.at[0], sem_x).wait()
    for p in range(4):
        wait(p)


def _expert(xs, idx, score, expert_down, expert_up):
    mesh = plsc.VectorSubcoreMesh(core_axis_name="c", subcore_axis_name="s")
    run = pl.kernel(
        _expert_body, mesh=mesh,
        compiler_params=pltpu.CompilerParams(needs_layout_passes=False),
        out_type=jax.ShapeDtypeStruct((T, D), jnp.float32),
        scratch_types=[
            pltpu.VMEM((TPW, HK), jnp.int32),
            pltpu.VMEM((TPW, HK), jnp.float32),
            pltpu.VMEM((2, D), jnp.float32),
            pltpu.VMEM((16, D), jnp.float32),
            pltpu.VMEM((16, D), jnp.float32),
            pltpu.VMEM((16, D), jnp.float32),
            pltpu.VMEM((16, D), jnp.float32),
            pltpu.VMEM((D,), jnp.float32),
            pltpu.VMEM((16, 16), jnp.float32),
            pltpu.SemaphoreType.DMA,
            pltpu.SemaphoreType.DMA,
            pltpu.SemaphoreType.DMA,
            pltpu.SemaphoreType.DMA,
            pltpu.SemaphoreType.DMA,
        ],
    )
    return run(xs, idx, score, expert_down, expert_up)


def kernel(x, Wq, keys, expert_down, expert_up):
    xs = x[0]
    keys1 = keys[:, :, 0, :]
    keys2 = keys[:, :, 1, :]
    idx, score = _routing(xs, Wq, keys1, keys2)
    out = _expert(xs, idx, score, expert_down, expert_up)
    return out[None]


# speedup vs baseline: 1.1937x; 1.0722x over previous
import functools

import jax
import jax.numpy as jnp
from jax import lax
from jax.experimental import pallas as pl
from jax.experimental.pallas import tpu as pltpu
from jax.experimental.pallas import tpu_sc as plsc

H = 4
K = 8
DK = 64
NKEYS = 128
D = 1024
T = 2048
TB = 256

NC = 2
NS = 16
NW = NC * NS
TPW = T // NW
HK = H * K

_NEG = float("-inf")


def _topk8_packed(s, nbits):
    iota = lax.broadcasted_iota(jnp.int32, s.shape, 1)
    mask = jnp.int32(~((1 << nbits) - 1))
    bits = lax.bitcast_convert_type(s, jnp.int32)
    cur = lax.bitcast_convert_type((bits & mask) | iota, jnp.float32)
    vals = []
    for _ in range(K):
        m = jnp.max(cur, axis=1, keepdims=True)
        vals.append(m)
        cur = jnp.where(cur == m, _NEG, cur)
    return jnp.concatenate(vals, axis=1)


def _routing_body(x_ref, wq_ref, k1_ref, k2_ref, idx_ref, score_ref):
    x = x_ref[...]
    q = lax.dot_general(x, wq_ref[...], (((1,), (1,)), ((), ())),
                        preferred_element_type=jnp.float32)
    for h in range(H):
        q1 = q[:, h * 2 * DK: h * 2 * DK + DK]
        q2 = q[:, h * 2 * DK + DK: (h + 1) * 2 * DK]
        sim1 = lax.dot_general(q1, k1_ref[h], (((1,), (1,)), ((), ())),
                               preferred_element_type=jnp.float32)
        sim2 = lax.dot_general(q2, k2_ref[h], (((1,), (1,)), ((), ())),
                               preferred_element_type=jnp.float32)
        p1 = _topk8_packed(sim1, 7)
        p2 = _topk8_packed(sim2, 7)
        i1 = lax.bitcast_convert_type(p1, jnp.int32) & (NKEYS - 1)
        i2 = lax.bitcast_convert_type(p2, jnp.int32) & (NKEYS - 1)
        all_s = jnp.concatenate([p1[:, i:i + 1] + p2 for i in range(K)], axis=1)
        pf = _topk8_packed(all_s, 6)
        pos = lax.bitcast_convert_type(pf, jnp.int32) & 63
        ipos = pos >> 3
        jpos = pos & 7
        iota8 = lax.broadcasted_iota(jnp.int32, i1.shape, 1)
        fi = []
        for s_ in range(K):
            sel1 = jnp.sum(jnp.where(iota8 == ipos[:, s_:s_ + 1], i1, 0),
                           axis=1, keepdims=True)
            sel2 = jnp.sum(jnp.where(iota8 == jpos[:, s_:s_ + 1], i2, 0),
                           axis=1, keepdims=True)
            fi.append(sel1 * NKEYS + sel2)
        idx_ref[:, h * K:(h + 1) * K] = jnp.concatenate(fi, axis=1)
        score_ref[:, h * K:(h + 1) * K] = jnp.maximum(pf, 0.0)


def _routing(xs, Wq, keys1, keys2):
    Th = xs.shape[0]
    grid = (Th // TB,)
    return pl.pallas_call(
        _routing_body,
        grid=grid,
        in_specs=[
            pl.BlockSpec((TB, D), lambda i: (i, 0)),
            pl.BlockSpec((2 * H * DK, D), lambda i: (0, 0)),
            pl.BlockSpec((H, NKEYS, DK), lambda i: (0, 0, 0)),
            pl.BlockSpec((H, NKEYS, DK), lambda i: (0, 0, 0)),
        ],
        out_specs=[
            pl.BlockSpec((TB, HK), lambda i: (i, 0)),
            pl.BlockSpec((TB, HK), lambda i: (i, 0)),
        ],
        out_shape=[
            jax.ShapeDtypeStruct((Th, HK), jnp.int32),
            jax.ShapeDtypeStruct((Th, HK), jnp.float32),
        ],
    )(xs, Wq, keys1, keys2)


def _expert_body(tpw, x_hbm, idx_hbm, score_hbm, down_hbm, up_hbm, out_hbm,
                 idx_v, score_v, xrow, d0, d1, u0, u1, obuf, red_v,
                 sem_d0, sem_d1, sem_u0, sem_u1, sem_x):
    TPW = tpw
    wid = lax.axis_index("s") * NC + lax.axis_index("c")
    base = wid * TPW
    pltpu.sync_copy(idx_hbm.at[pl.ds(base, TPW)], idx_v)
    pltpu.sync_copy(score_hbm.at[pl.ds(base, TPW)], score_v)

    zeros16 = jnp.zeros((16,), jnp.float32)
    bufs = (d0, d1, u0, u1)
    sems = (sem_d0, sem_d1, sem_u0, sem_u1)
    tabs = (down_hbm, down_hbm, up_hbm, up_hbm)

    def issue(i, p):
        half = (p % 2) * 16
        pltpu.async_copy(
            tabs[p].at[idx_v.at[i, pl.ds(half, 16)]], bufs[p], sems[p])

    def wait(p):
        pltpu.make_async_copy(
            tabs[p].at[idx_v.at[0, pl.ds((p % 2) * 16, 16)]],
            bufs[p], sems[p]).wait()

    def dots32(par, i):
        def dot_step(c, accs):
            xv = xrow[par, pl.ds(c, 16)]
            lo = [accs[r] + xv * d0[r, pl.ds(c, 16)] for r in range(16)]
            hi = [accs[16 + r] + xv * d1[r, pl.ds(c, 16)] for r in range(16)]
            return tuple(lo + hi)

        accs = plsc.parallel_loop(
            0, D, step=16, unroll=4, carry=(zeros16,) * HK)(dot_step)
        for r in range(HK):
            red_v[r] = accs[r]
        iota16 = lax.iota(jnp.int32, 16)
        hvs = []
        for half in range(2):
            hv = zeros16
            for c in range(16):
                hv = hv + plsc.load_gather(
                    red_v, [iota16 + half * 16, jnp.full((16,), c, jnp.int32)])
            hv = hv / (1.0 + jnp.exp(-hv))
            hvs.append(hv * score_v[i, pl.ds(half * 16, 16)])
        return hvs

    def accum32(hv0, hv1):
        hs0 = [hv0[r] for r in range(16)]
        hs1 = [hv1[r] for r in range(16)]

        def out_step(c, _):
            acc = zeros16
            for r in range(16):
                acc = acc + hs0[r] * u0[r, pl.ds(c, 16)]
            for r in range(16):
                acc = acc + hs1[r] * u1[r, pl.ds(c, 16)]
            obuf[pl.ds(c, 16)] = acc
            return _

        plsc.parallel_loop(0, D, step=16, unroll=4, carry=jnp.int32(0))(
            out_step)

    pltpu.async_copy(x_hbm.at[base], xrow.at[0], sem_x)
    for p in range(4):
        issue(jnp.int32(0), p)

    def pair_step(ii, carry):
        for par in range(2):
            i = ii * 2 + par
            t = base + i
            inext = jnp.minimum(i + 1, TPW - 1)
            pltpu.make_async_copy(x_hbm.at[t], xrow.at[par], sem_x).wait()
            wait(0)
            wait(1)
            hv0, hv1 = dots32(par, i)
            wait(2)
            wait(3)
            accum32(hv0, hv1)
            pltpu.async_copy(x_hbm.at[base + inext], xrow.at[1 - par], sem_x)
            for p in range(4):
                issue(inext, p)
            pltpu.sync_copy(obuf, out_hbm.at[t])
        return carry

    lax.fori_loop(0, TPW // 2, pair_step, 0)
    pltpu.make_async_copy(x_hbm.at[base], xrow.at[0], sem_x).wait()
    for p in range(4):
        wait(p)


def _expert(xs, idx, score, expert_down, expert_up):
    Th = xs.shape[0]
    tpw = Th // NW
    mesh = plsc.VectorSubcoreMesh(core_axis_name="c", subcore_axis_name="s")
    run = pl.kernel(
        functools.partial(_expert_body, tpw), mesh=mesh,
        compiler_params=pltpu.CompilerParams(needs_layout_passes=False),
        out_type=jax.ShapeDtypeStruct((Th, D), jnp.float32),
        scratch_types=[
            pltpu.VMEM((tpw, HK), jnp.int32),
            pltpu.VMEM((tpw, HK), jnp.float32),
            pltpu.VMEM((2, D), jnp.float32),
            pltpu.VMEM((16, D), jnp.float32),
            pltpu.VMEM((16, D), jnp.float32),
            pltpu.VMEM((16, D), jnp.float32),
            pltpu.VMEM((16, D), jnp.float32),
            pltpu.VMEM((D,), jnp.float32),
            pltpu.VMEM((16, 16), jnp.float32),
            pltpu.SemaphoreType.DMA,
            pltpu.SemaphoreType.DMA,
            pltpu.SemaphoreType.DMA,
            pltpu.SemaphoreType.DMA,
            pltpu.SemaphoreType.DMA,
        ],
    )
    return run(xs, idx, score, expert_down, expert_up)


def kernel(x, Wq, keys, expert_down, expert_up):
    xs = x[0]
    keys1 = keys[:, :, 0, :]
    keys2 = keys[:, :, 1, :]
    nchunk = 2
    ch = T // nchunk
    routed = [
        _routing(xs[ci * ch:(ci + 1) * ch], Wq, keys1, keys2)
        for ci in range(nchunk)
    ]
    outs = [
        _expert(xs[ci * ch:(ci + 1) * ch], idx, score,
                expert_down, expert_up)
        for ci, (idx, score) in enumerate(routed)
    ]
    return jnp.concatenate(outs, axis=0)[None]


# speedup vs baseline: 1.2124x; 1.0156x over previous
import functools

import jax
import jax.numpy as jnp
from jax import lax
from jax.experimental import pallas as pl
from jax.experimental.pallas import tpu as pltpu
from jax.experimental.pallas import tpu_sc as plsc

H = 4
K = 8
DK = 64
NKEYS = 128
D = 1024
T = 2048
TB = 256

NC = 2
NS = 16
NW = NC * NS
TPW = T // NW
HK = H * K

_NEG = float("-inf")


def _topk8_packed(s, nbits):
    iota = lax.broadcasted_iota(jnp.int32, s.shape, 1)
    mask = jnp.int32(~((1 << nbits) - 1))
    bits = lax.bitcast_convert_type(s, jnp.int32)
    cur = lax.bitcast_convert_type((bits & mask) | iota, jnp.float32)
    vals = []
    for _ in range(K):
        m = jnp.max(cur, axis=1, keepdims=True)
        vals.append(m)
        cur = jnp.where(cur == m, _NEG, cur)
    return jnp.concatenate(vals, axis=1)


def _routing_body(x_ref, wq_ref, k1_ref, k2_ref, idx_ref, score_ref):
    x = x_ref[...]
    q = lax.dot_general(x, wq_ref[...], (((1,), (1,)), ((), ())),
                        preferred_element_type=jnp.float32)
    for h in range(H):
        q1 = q[:, h * 2 * DK: h * 2 * DK + DK]
        q2 = q[:, h * 2 * DK + DK: (h + 1) * 2 * DK]
        sim1 = lax.dot_general(q1, k1_ref[h], (((1,), (1,)), ((), ())),
                               preferred_element_type=jnp.float32)
        sim2 = lax.dot_general(q2, k2_ref[h], (((1,), (1,)), ((), ())),
                               preferred_element_type=jnp.float32)
        p1 = _topk8_packed(sim1, 7)
        p2 = _topk8_packed(sim2, 7)
        i1 = lax.bitcast_convert_type(p1, jnp.int32) & (NKEYS - 1)
        i2 = lax.bitcast_convert_type(p2, jnp.int32) & (NKEYS - 1)
        all_s = jnp.concatenate([p1[:, i:i + 1] + p2 for i in range(K)], axis=1)
        pf = _topk8_packed(all_s, 6)
        pos = lax.bitcast_convert_type(pf, jnp.int32) & 63
        ipos = pos >> 3
        jpos = pos & 7
        iota8 = lax.broadcasted_iota(jnp.int32, i1.shape, 1)
        fi = []
        for s_ in range(K):
            sel1 = jnp.sum(jnp.where(iota8 == ipos[:, s_:s_ + 1], i1, 0),
                           axis=1, keepdims=True)
            sel2 = jnp.sum(jnp.where(iota8 == jpos[:, s_:s_ + 1], i2, 0),
                           axis=1, keepdims=True)
            fi.append(sel1 * NKEYS + sel2)
        idx_ref[:, h * K:(h + 1) * K] = jnp.concatenate(fi, axis=1)
        score_ref[:, h * K:(h + 1) * K] = jnp.maximum(pf, 0.0)


def _routing(xs, Wq, keys1, keys2):
    Th = xs.shape[0]
    grid = (Th // TB,)
    return pl.pallas_call(
        _routing_body,
        grid=grid,
        in_specs=[
            pl.BlockSpec((TB, D), lambda i: (i, 0)),
            pl.BlockSpec((2 * H * DK, D), lambda i: (0, 0)),
            pl.BlockSpec((H, NKEYS, DK), lambda i: (0, 0, 0)),
            pl.BlockSpec((H, NKEYS, DK), lambda i: (0, 0, 0)),
        ],
        out_specs=[
            pl.BlockSpec((TB, HK), lambda i: (i, 0)),
            pl.BlockSpec((TB, HK), lambda i: (i, 0)),
        ],
        out_shape=[
            jax.ShapeDtypeStruct((Th, HK), jnp.int32),
            jax.ShapeDtypeStruct((Th, HK), jnp.float32),
        ],
    )(xs, Wq, keys1, keys2)


def _expert_body(tpw, x_hbm, idx_hbm, score_hbm, down_hbm, up_hbm, out_hbm,
                 idx_v, score_v, xrow, d0, d1, u0, u1, obuf, red_v,
                 sem_d0, sem_d1, sem_u0, sem_u1, sem_x):
    TPW = tpw
    wid = lax.axis_index("s") * NC + lax.axis_index("c")
    base = wid * TPW
    pltpu.sync_copy(idx_hbm.at[pl.ds(base, TPW)], idx_v)
    pltpu.sync_copy(score_hbm.at[pl.ds(base, TPW)], score_v)

    zeros16 = jnp.zeros((16,), jnp.float32)
    bufs = (d0, d1, u0, u1)
    sems = (sem_d0, sem_d1, sem_u0, sem_u1)
    tabs = (down_hbm, down_hbm, up_hbm, up_hbm)

    def issue(i, p):
        half = (p % 2) * 16
        pltpu.async_copy(
            tabs[p].at[idx_v.at[i, pl.ds(half, 16)]], bufs[p], sems[p])

    def wait(p):
        pltpu.make_async_copy(
            tabs[p].at[idx_v.at[0, pl.ds((p % 2) * 16, 16)]],
            bufs[p], sems[p]).wait()

    def dots32(par, i):
        def dot_step(c, accs):
            xv = xrow[par, pl.ds(c, 16)]
            lo = [accs[r] + xv * d0[r, pl.ds(c, 16)] for r in range(16)]
            hi = [accs[16 + r] + xv * d1[r, pl.ds(c, 16)] for r in range(16)]
            return tuple(lo + hi)

        accs = plsc.parallel_loop(
            0, D, step=16, unroll=4, carry=(zeros16,) * HK)(dot_step)
        for r in range(HK):
            red_v[r] = accs[r]
        iota16 = lax.iota(jnp.int32, 16)
        hvs = []
        for half in range(2):
            hv = zeros16
            for c in range(16):
                hv = hv + plsc.load_gather(
                    red_v, [iota16 + half * 16, jnp.full((16,), c, jnp.int32)])
            hv = hv / (1.0 + jnp.exp(-hv))
            hvs.append(hv * score_v[i, pl.ds(half * 16, 16)])
        return hvs

    def accum32(hv0, hv1):
        hs0 = [hv0[r] for r in range(16)]
        hs1 = [hv1[r] for r in range(16)]

        def out_step(c, _):
            acc = zeros16
            for r in range(16):
                acc = acc + hs0[r] * u0[r, pl.ds(c, 16)]
            for r in range(16):
                acc = acc + hs1[r] * u1[r, pl.ds(c, 16)]
            obuf[pl.ds(c, 16)] = acc
            return _

        plsc.parallel_loop(0, D, step=16, unroll=4, carry=jnp.int32(0))(
            out_step)

    pltpu.async_copy(x_hbm.at[base], xrow.at[0], sem_x)
    for p in range(4):
        issue(jnp.int32(0), p)

    def pair_step(ii, carry):
        for par in range(2):
            i = ii * 2 + par
            t = base + i
            inext = jnp.minimum(i + 1, TPW - 1)
            pltpu.make_async_copy(x_hbm.at[t], xrow.at[par], sem_x).wait()
            wait(0)
            wait(1)
            hv0, hv1 = dots32(par, i)
            wait(2)
            wait(3)
            accum32(hv0, hv1)
            pltpu.async_copy(x_hbm.at[base + inext], xrow.at[1 - par], sem_x)
            for p in range(4):
                issue(inext, p)
            pltpu.sync_copy(obuf, out_hbm.at[t])
        return carry

    lax.fori_loop(0, TPW // 2, pair_step, 0)
    pltpu.make_async_copy(x_hbm.at[base], xrow.at[0], sem_x).wait()
    for p in range(4):
        wait(p)


def _expert(xs, idx, score, expert_down, expert_up):
    Th = xs.shape[0]
    tpw = Th // NW
    mesh = plsc.VectorSubcoreMesh(core_axis_name="c", subcore_axis_name="s")
    run = pl.kernel(
        functools.partial(_expert_body, tpw), mesh=mesh,
        compiler_params=pltpu.CompilerParams(needs_layout_passes=False),
        out_type=jax.ShapeDtypeStruct((Th, D), jnp.float32),
        scratch_types=[
            pltpu.VMEM((tpw, HK), jnp.int32),
            pltpu.VMEM((tpw, HK), jnp.float32),
            pltpu.VMEM((2, D), jnp.float32),
            pltpu.VMEM((16, D), jnp.float32),
            pltpu.VMEM((16, D), jnp.float32),
            pltpu.VMEM((16, D), jnp.float32),
            pltpu.VMEM((16, D), jnp.float32),
            pltpu.VMEM((D,), jnp.float32),
            pltpu.VMEM((16, 16), jnp.float32),
            pltpu.SemaphoreType.DMA,
            pltpu.SemaphoreType.DMA,
            pltpu.SemaphoreType.DMA,
            pltpu.SemaphoreType.DMA,
            pltpu.SemaphoreType.DMA,
        ],
    )
    return run(xs, idx, score, expert_down, expert_up)


def kernel(x, Wq, keys, expert_down, expert_up):
    xs = x[0]
    keys1 = keys[:, :, 0, :]
    keys2 = keys[:, :, 1, :]
    nchunk = 4
    ch = T // nchunk
    routed = [
        _routing(xs[ci * ch:(ci + 1) * ch], Wq, keys1, keys2)
        for ci in range(nchunk)
    ]
    outs = [
        _expert(xs[ci * ch:(ci + 1) * ch], idx, score,
                expert_down, expert_up)
        for ci, (idx, score) in enumerate(routed)
    ]
    return jnp.concatenate(outs, axis=0)[None]
